# baseline (speedup 1.0000x reference)
import jax
import jax.numpy as jnp
from jax import lax
from jax.experimental import pallas as pl
from jax.experimental.pallas import tpu as pltpu
from jax.experimental.pallas import tpu_sc as plsc

_NC, _NS = 2, 16
_NW = _NC * _NS
_RG_PER_W = 4
_CC = 8
_CHUNK = 2048
_IW = _CHUNK // 128


def _body(p_hbm, m_hbm, sb_hbm, par_hbm, out_hbm, p_v, m_v, o_v, sb_v, par_v):
    c = lax.axis_index("c")
    s = lax.axis_index("s")
    wid = s * _NC + c
    pltpu.sync_copy(
        sb_hbm.at[pl.ds(pl.multiple_of(wid * 32, 32), 32), :, :], sb_v)
    pltpu.sync_copy(par_hbm, par_v)
    alpha = par_v[pl.ds(0, 16)][0]
    neg_inf = jnp.float32(-1e9)
    lanes = lax.iota(jnp.int32, 16)
    zero_v = jnp.zeros((16,), jnp.int32)
    jidx = [[64 * h + 4 * lanes + r for r in range(4)] for h in range(2)]

    def _slices(rg, cc):
        n0 = rg * 8
        b = n0 // 64
        t0 = n0 % 64
        i0 = pl.multiple_of(cc * _IW, _IW)
        k0 = pl.multiple_of(cc * _CHUNK, _CHUNK)
        kw0 = pl.multiple_of(cc * (_CHUNK // 4), _CHUNK // 4)
        to = pl.multiple_of(t0, 8)
        psrc = p_hbm.at[b, pl.ds(t0, 8), pl.ds(i0, _IW), :]
        msrc = m_hbm.at[b, pl.ds(to, 8), pl.ds(kw0, _CHUNK // 4)]
        odst = out_hbm.at[b, pl.ds(to, 8), pl.ds(k0, _CHUNK)]
        return psrc, msrc, odst

    def _compute(slot, local0):
        def row(rr, carry):
            rrv = jnp.full((16,), rr, jnp.int32)
            slotv = jnp.full((16,), slot, jnp.int32)
            sbrv = sb_v[local0 + rr, 0, pl.ds(0, 16)]
            for ii in range(_IW):
                for h in range(2):
                    w = m_v[slot, rr, pl.ds(ii * 32 + h * 16, 16)]
                    iiv = jnp.full((16,), ii, jnp.int32)
                    for r in range(4):
                        keep = (w << (31 - 8 * r)) < 0
                        v = plsc.load_gather(
                            p_v, [slotv, rrv, iiv, jidx[h][r]])
                        res = jnp.where(keep, sbrv - alpha * v, neg_inf)
                        plsc.store_scatter(
                            o_v, [slotv, rrv, jidx[h][r] + (ii * 128)], res)
            return carry

        lax.fori_loop(0, 8, row, 0)

    for rg_i in range(_RG_PER_W):
        rg = wid * _RG_PER_W + rg_i
        local0 = 8 * rg_i

        def step(cc, carry, rg=rg, local0=local0):
            psrc, msrc, odst = _slices(rg, cc)
            pltpu.sync_copy(psrc, p_v.at[0])
            pltpu.sync_copy(msrc, m_v.at[0])
            _compute(0, local0)
            pltpu.sync_copy(o_v.at[0], odst)
            return carry

        lax.fori_loop(0, _CC, step, 0)


def kernel(pairwise_tti, combined_mask, state_bias, alpha, bias):
    b, t, i, j = pairwise_tti.shape
    m32 = combined_mask.reshape(b, t, i * j).view(jnp.uint8).view(jnp.int32)
    sbx = jnp.broadcast_to(
        ((state_bias + bias).reshape(b * t))[:, None, None], (b * t, 1, 16))
    params = jnp.broadcast_to(alpha.reshape(1), (16,))
    mesh = plsc.VectorSubcoreMesh(core_axis_name="c", subcore_axis_name="s",
                                  num_cores=_NC, num_subcores=_NS)
    f = pl.kernel(
        _body,
        out_type=jax.ShapeDtypeStruct((b, t, i * j), jnp.float32),
        mesh=mesh,
        compiler_params=pltpu.CompilerParams(needs_layout_passes=False),
        scratch_types=[
            pltpu.VMEM((2, 8, _IW, 128), jnp.float32),
            pltpu.VMEM((2, 8, _CHUNK // 4), jnp.int32),
            pltpu.VMEM((2, 8, _CHUNK), jnp.float32),
            pltpu.VMEM((32, 1, 16), jnp.float32),
            pltpu.VMEM((16,), jnp.float32),
        ],
    )
    return f(pairwise_tti, m32, sbx, params)


# speedup vs baseline: 1.1775x; 1.1775x over previous
import jax
import jax.numpy as jnp
from jax import lax
from jax.experimental import pallas as pl
from jax.experimental.pallas import tpu as pltpu
from jax.experimental.pallas import tpu_sc as plsc

_NC, _NS = 2, 16
_NW = _NC * _NS
_RG_PER_W = 4
_CC = 8
_CHUNK = 2048
_IW = _CHUNK // 128


def _body(p_hbm, m_hbm, sb_hbm, par_hbm, out_hbm, p_v, m_v, o_v, sb_v, par_v,
          sem_in, sem_out):
    c = lax.axis_index("c")
    s = lax.axis_index("s")
    wid = s * _NC + c
    pltpu.sync_copy(
        sb_hbm.at[pl.ds(pl.multiple_of(wid * 32, 32), 32), :, :], sb_v)
    pltpu.sync_copy(par_hbm, par_v)
    alpha = par_v[pl.ds(0, 16)][0]
    neg_inf = jnp.float32(-1e9)
    lanes = lax.iota(jnp.int32, 16)
    zero_v = jnp.zeros((16,), jnp.int32)
    jidx = [[64 * h + 4 * lanes + r for r in range(4)] for h in range(2)]

    def _slices(rg, cc):
        n0 = rg * 8
        b = n0 // 64
        t0 = n0 % 64
        i0 = pl.multiple_of(cc * _IW, _IW)
        k0 = pl.multiple_of(cc * _CHUNK, _CHUNK)
        kw0 = pl.multiple_of(cc * (_CHUNK // 4), _CHUNK // 4)
        to = pl.multiple_of(t0, 8)
        psrc = p_hbm.at[b, pl.ds(t0, 8), pl.ds(i0, _IW), :]
        msrc = m_hbm.at[b, pl.ds(to, 8), pl.ds(kw0, _CHUNK // 4)]
        odst = out_hbm.at[b, pl.ds(to, 8), pl.ds(k0, _CHUNK)]
        return psrc, msrc, odst

    def _compute(slot, local0):
        def row(rr, carry):
            rrv = jnp.full((16,), rr, jnp.int32)
            slotv = jnp.full((16,), slot, jnp.int32)
            sbrv = sb_v[local0 + rr, 0, pl.ds(0, 16)]
            for ii in range(_IW):
                for h in range(2):
                    w = m_v[slot, rr, pl.ds(ii * 32 + h * 16, 16)]
                    iiv = jnp.full((16,), ii, jnp.int32)
                    for r in range(4):
                        keep = (w << (31 - 8 * r)) < 0
                        v = plsc.load_gather(
                            p_v, [slotv, rrv, iiv, jidx[h][r]])
                        res = jnp.where(keep, sbrv - alpha * v, neg_inf)
                        plsc.store_scatter(
                            o_v, [slotv, rrv, jidx[h][r] + (ii * 128)], res)
            return carry

        lax.fori_loop(0, 8, row, 0)

    def _issue_in(rg, cc, slot):
        psrc, msrc, _ = _slices(rg, cc)
        pltpu.async_copy(psrc, p_v.at[slot], sem_in)
        pltpu.async_copy(msrc, m_v.at[slot], sem_in)

    def _wait_in(rg, cc, slot):
        psrc, msrc, _ = _slices(rg, cc)
        pltpu.make_async_copy(psrc, p_v.at[slot], sem_in).wait()
        pltpu.make_async_copy(msrc, m_v.at[slot], sem_in).wait()

    def _issue_out(rg, cc, slot):
        _, _, odst = _slices(rg, cc)
        pltpu.async_copy(o_v.at[slot], odst, sem_out)

    def _wait_out(rg, cc, slot):
        _, _, odst = _slices(rg, cc)
        pltpu.make_async_copy(o_v.at[slot], odst, sem_out).wait()

    for rg_i in range(_RG_PER_W):
        rg = wid * _RG_PER_W + rg_i
        local0 = 8 * rg_i
        _issue_in(rg, 0, 0)

        def step(cc, carry, rg=rg, local0=local0):
            slot = cc % 2

            @pl.when(cc + 1 < _CC)
            def _():
                _issue_in(rg, cc + 1, (cc + 1) % 2)

            _wait_in(rg, cc, slot)

            @pl.when(cc >= 2)
            def _():
                _wait_out(rg, cc - 2, slot)

            _compute(slot, local0)
            _issue_out(rg, cc, slot)
            return carry

        lax.fori_loop(0, _CC, step, 0)
        _wait_out(rg, _CC - 2, 0)
        _wait_out(rg, _CC - 1, 1)


def kernel(pairwise_tti, combined_mask, state_bias, alpha, bias):
    b, t, i, j = pairwise_tti.shape
    m32 = combined_mask.reshape(b, t, i * j).view(jnp.uint8).view(jnp.int32)
    sbx = jnp.broadcast_to(
        ((state_bias + bias).reshape(b * t))[:, None, None], (b * t, 1, 16))
    params = jnp.broadcast_to(alpha.reshape(1), (16,))
    mesh = plsc.VectorSubcoreMesh(core_axis_name="c", subcore_axis_name="s",
                                  num_cores=_NC, num_subcores=_NS)
    f = pl.kernel(
        _body,
        out_type=jax.ShapeDtypeStruct((b, t, i * j), jnp.float32),
        mesh=mesh,
        compiler_params=pltpu.CompilerParams(needs_layout_passes=False),
        scratch_types=[
            pltpu.VMEM((2, 8, _IW, 128), jnp.float32),
            pltpu.VMEM((2, 8, _CHUNK // 4), jnp.int32),
            pltpu.VMEM((2, 8, _CHUNK), jnp.float32),
            pltpu.VMEM((32, 1, 16), jnp.float32),
            pltpu.VMEM((16,), jnp.float32),
            pltpu.SemaphoreType.DMA,
            pltpu.SemaphoreType.DMA,
        ],
    )
    return f(pairwise_tti, m32, sbx, params)


# speedup vs baseline: 1.1853x; 1.0066x over previous
import jax
import jax.numpy as jnp
from jax import lax
from jax.experimental import pallas as pl
from jax.experimental.pallas import tpu as pltpu
from jax.experimental.pallas import tpu_sc as plsc

_NC, _NS = 2, 16
_NW = _NC * _NS
_RG_PER_W = 4
_CC = 8
_CHUNK = 2048
_IW = _CHUNK // 128


def _body(p_hbm, m_hbm, sb_hbm, par_hbm, out_hbm, p_v, m_v, o_v, sb_v, par_v,
          sem_in, sem_out):
    c = lax.axis_index("c")
    s = lax.axis_index("s")
    wid = s * _NC + c
    pltpu.sync_copy(
        sb_hbm.at[pl.ds(pl.multiple_of(wid * 32, 32), 32), :, :], sb_v)
    pltpu.sync_copy(par_hbm, par_v)
    alpha = par_v[pl.ds(0, 16)][0]
    neg_inf = jnp.float32(-1e9)

    def _slices(rg, cc):
        n0 = rg * 8
        b = n0 // 64
        t0 = n0 % 64
        i0 = pl.multiple_of(cc * _IW, _IW)
        k0 = pl.multiple_of(cc * _CHUNK, _CHUNK)
        kw0 = pl.multiple_of(cc * (_CHUNK // 4), _CHUNK // 4)
        to = pl.multiple_of(t0, 8)
        psrc = p_hbm.at[b, pl.ds(t0, 8), pl.ds(i0, _IW), :]
        msrc = m_hbm.at[b, pl.ds(to, 8), pl.ds(kw0, _CHUNK // 4)]
        odst = out_hbm.at[b, pl.ds(to, 8), pl.ds(k0, _CHUNK)]
        return psrc, msrc, odst

    def _compute(slot, local0):
        def row(rr, carry):
            sbrv = sb_v[local0 + rr, 0, pl.ds(0, 16)]
            for ii in range(_IW):
                for h in range(2):
                    w = m_v[slot, rr, pl.ds(ii * 32 + h * 16, 16)]
                    for r in range(4):
                        keep = (w << (31 - 8 * r)) < 0
                        jo = 64 * h + 16 * r
                        v = p_v[slot, rr, ii, pl.ds(jo, 16)]
                        o_v[slot, rr, pl.ds(ii * 128 + jo, 16)] = jnp.where(
                            keep, sbrv - alpha * v, neg_inf)
            return carry

        lax.fori_loop(0, 8, row, 0)

    def _issue_in(rg, cc, slot):
        psrc, msrc, _ = _slices(rg, cc)
        pltpu.async_copy(psrc, p_v.at[slot], sem_in)
        pltpu.async_copy(msrc, m_v.at[slot], sem_in)

    def _wait_in(rg, cc, slot):
        psrc, msrc, _ = _slices(rg, cc)
        pltpu.make_async_copy(psrc, p_v.at[slot], sem_in).wait()
        pltpu.make_async_copy(msrc, m_v.at[slot], sem_in).wait()

    def _issue_out(rg, cc, slot):
        _, _, odst = _slices(rg, cc)
        pltpu.async_copy(o_v.at[slot], odst, sem_out)

    def _wait_out(rg, cc, slot):
        _, _, odst = _slices(rg, cc)
        pltpu.make_async_copy(o_v.at[slot], odst, sem_out).wait()

    for rg_i in range(_RG_PER_W):
        rg = wid * _RG_PER_W + rg_i
        local0 = 8 * rg_i
        _issue_in(rg, 0, 0)

        def step(cc, carry, rg=rg, local0=local0):
            slot = cc % 2

            @pl.when(cc + 1 < _CC)
            def _():
                _issue_in(rg, cc + 1, (cc + 1) % 2)

            _wait_in(rg, cc, slot)

            @pl.when(cc >= 2)
            def _():
                _wait_out(rg, cc - 2, slot)

            _compute(slot, local0)
            _issue_out(rg, cc, slot)
            return carry

        lax.fori_loop(0, _CC, step, 0)
        _wait_out(rg, _CC - 2, 0)
        _wait_out(rg, _CC - 1, 1)


def kernel(pairwise_tti, combined_mask, state_bias, alpha, bias):
    b, t, i, j = pairwise_tti.shape
    m32 = (combined_mask.reshape(b, t, (i * j) // 64, 4, 16)
           .transpose(0, 1, 2, 4, 3)
           .view(jnp.uint8).view(jnp.int32).reshape(b, t, (i * j) // 4))
    sbx = jnp.broadcast_to(
        ((state_bias + bias).reshape(b * t))[:, None, None], (b * t, 1, 16))
    params = jnp.broadcast_to(alpha.reshape(1), (16,))
    mesh = plsc.VectorSubcoreMesh(core_axis_name="c", subcore_axis_name="s",
                                  num_cores=_NC, num_subcores=_NS)
    f = pl.kernel(
        _body,
        out_type=jax.ShapeDtypeStruct((b, t, i * j), jnp.float32),
        mesh=mesh,
        compiler_params=pltpu.CompilerParams(needs_layout_passes=False),
        scratch_types=[
            pltpu.VMEM((2, 8, _IW, 128), jnp.float32),
            pltpu.VMEM((2, 8, _CHUNK // 4), jnp.int32),
            pltpu.VMEM((2, 8, _CHUNK), jnp.float32),
            pltpu.VMEM((32, 1, 16), jnp.float32),
            pltpu.VMEM((16,), jnp.float32),
            pltpu.SemaphoreType.DMA,
            pltpu.SemaphoreType.DMA,
        ],
    )
    return f(pairwise_tti, m32, sbx, params)


# speedup vs baseline: 1.8604x; 1.5696x over previous
import jax
import jax.numpy as jnp
from jax import lax
from jax.experimental import pallas as pl
from jax.experimental.pallas import tpu as pltpu
from jax.experimental.pallas import tpu_sc as plsc

_NC, _NS = 2, 16
_NW = _NC * _NS
_RG_PER_W = 4
_CC = 8
_CHUNK = 2048
_IW = _CHUNK // 128


def _body(p_hbm, m_hbm, sb_hbm, par_hbm, out_hbm, p_v, m_v, o_v, sb_v, par_v,
          sem_in, sem_out):
    c = lax.axis_index("c")
    s = lax.axis_index("s")
    wid = s * _NC + c
    pltpu.sync_copy(
        sb_hbm.at[pl.ds(pl.multiple_of(wid * 32, 32), 32), :, :], sb_v)
    pltpu.sync_copy(par_hbm, par_v)
    alpha = par_v[pl.ds(0, 16)][0]
    neg_inf = jnp.float32(-1e9)

    def _slices(rg, cc):
        n0 = rg * 8
        b = n0 // 64
        t0 = n0 % 64
        i0 = pl.multiple_of(cc * _IW, _IW)
        k0 = pl.multiple_of(cc * _CHUNK, _CHUNK)
        kw0 = pl.multiple_of(cc * (_CHUNK // 4), _CHUNK // 4)
        to = pl.multiple_of(t0, 8)
        psrc = p_hbm.at[b, pl.ds(t0, 8), pl.ds(i0, _IW), :]
        msrc = m_hbm.at[b, pl.ds(to, 8), pl.ds(kw0, _CHUNK // 4)]
        odst = out_hbm.at[b, pl.ds(to, 8), pl.ds(k0, _CHUNK)]
        return psrc, msrc, odst

    def _compute(slot, local0):
        def row(rr, carry):
            sbrv = sb_v[local0 + rr, 0, pl.ds(0, 16)]

            @plsc.parallel_loop(0, _IW, unroll=2)
            def _(ii):
                for h in range(2):
                    w = m_v[slot, rr, pl.ds(ii * 32 + h * 16, 16)]
                    for r in range(4):
                        keep = (w << (31 - 8 * r)) < 0
                        jo = 64 * h + 16 * r
                        v = p_v[slot, rr, ii, pl.ds(jo, 16)]
                        o_v[slot, rr, pl.ds(ii * 128 + jo, 16)] = jnp.where(
                            keep, sbrv - alpha * v, neg_inf)

            return carry

        lax.fori_loop(0, 8, row, 0)

    def _issue_in(rg, cc, slot):
        psrc, msrc, _ = _slices(rg, cc)
        pltpu.async_copy(psrc, p_v.at[slot], sem_in)
        pltpu.async_copy(msrc, m_v.at[slot], sem_in)

    def _wait_in(rg, cc, slot):
        psrc, msrc, _ = _slices(rg, cc)
        pltpu.make_async_copy(psrc, p_v.at[slot], sem_in).wait()
        pltpu.make_async_copy(msrc, m_v.at[slot], sem_in).wait()

    def _issue_out(rg, cc, slot):
        _, _, odst = _slices(rg, cc)
        pltpu.async_copy(o_v.at[slot], odst, sem_out)

    def _wait_out(rg, cc, slot):
        _, _, odst = _slices(rg, cc)
        pltpu.make_async_copy(o_v.at[slot], odst, sem_out).wait()

    for rg_i in range(_RG_PER_W):
        rg = wid * _RG_PER_W + rg_i
        local0 = 8 * rg_i
        _issue_in(rg, 0, 0)

        def step(cc, carry, rg=rg, local0=local0):
            slot = cc % 2

            @pl.when(cc + 1 < _CC)
            def _():
                _issue_in(rg, cc + 1, (cc + 1) % 2)

            _wait_in(rg, cc, slot)

            @pl.when(cc >= 2)
            def _():
                _wait_out(rg, cc - 2, slot)

            _compute(slot, local0)
            _issue_out(rg, cc, slot)
            return carry

        lax.fori_loop(0, _CC, step, 0)
        _wait_out(rg, _CC - 2, 0)
        _wait_out(rg, _CC - 1, 1)


def kernel(pairwise_tti, combined_mask, state_bias, alpha, bias):
    b, t, i, j = pairwise_tti.shape
    m32 = (combined_mask.reshape(b, t, (i * j) // 64, 4, 16)
           .transpose(0, 1, 2, 4, 3)
           .view(jnp.uint8).view(jnp.int32).reshape(b, t, (i * j) // 4))
    sbx = jnp.broadcast_to(
        ((state_bias + bias).reshape(b * t))[:, None, None], (b * t, 1, 16))
    params = jnp.broadcast_to(alpha.reshape(1), (16,))
    mesh = plsc.VectorSubcoreMesh(core_axis_name="c", subcore_axis_name="s",
                                  num_cores=_NC, num_subcores=_NS)
    f = pl.kernel(
        _body,
        out_type=jax.ShapeDtypeStruct((b, t, i * j), jnp.float32),
        mesh=mesh,
        compiler_params=pltpu.CompilerParams(needs_layout_passes=False),
        scratch_types=[
            pltpu.VMEM((2, 8, _IW, 128), jnp.float32),
            pltpu.VMEM((2, 8, _CHUNK // 4), jnp.int32),
            pltpu.VMEM((2, 8, _CHUNK), jnp.float32),
            pltpu.VMEM((32, 1, 16), jnp.float32),
            pltpu.VMEM((16,), jnp.float32),
            pltpu.SemaphoreType.DMA,
            pltpu.SemaphoreType.DMA,
        ],
    )
    return f(pairwise_tti, m32, sbx, params)


# speedup vs baseline: 1.8756x; 1.0082x over previous
import jax
import jax.numpy as jnp
from jax import lax
from jax.experimental import pallas as pl
from jax.experimental.pallas import tpu as pltpu
from jax.experimental.pallas import tpu_sc as plsc

_NC, _NS = 2, 16
_NW = _NC * _NS
_RG_PER_W = 4
_CC = 8
_CHUNK = 2048
_IW = _CHUNK // 128


def _body(p_hbm, m_hbm, sb_hbm, par_hbm, out_hbm, p_v, m_v, o_v, sb_v, par_v,
          sem_in, sem_out):
    c = lax.axis_index("c")
    s = lax.axis_index("s")
    wid = s * _NC + c
    pltpu.sync_copy(
        sb_hbm.at[pl.ds(pl.multiple_of(wid * 32, 32), 32), :, :], sb_v)
    pltpu.sync_copy(par_hbm, par_v)
    alpha = par_v[pl.ds(0, 16)][0]
    neg_inf = jnp.float32(-1e9)

    def _slices(rg, cc):
        n0 = rg * 8
        b = n0 // 64
        t0 = n0 % 64
        i0 = pl.multiple_of(cc * _IW, _IW)
        k0 = pl.multiple_of(cc * _CHUNK, _CHUNK)
        kw0 = pl.multiple_of(cc * (_CHUNK // 4), _CHUNK // 4)
        to = pl.multiple_of(t0, 8)
        psrc = p_hbm.at[b, pl.ds(t0, 8), pl.ds(i0, _IW), :]
        msrc = m_hbm.at[b, pl.ds(to, 8), pl.ds(kw0, _CHUNK // 4)]
        odst = out_hbm.at[b, pl.ds(to, 8), pl.ds(k0, _CHUNK)]
        return psrc, msrc, odst

    def _compute(slot, local0):
        def row(rr, carry):
            sbrv = sb_v[local0 + rr, 0, pl.ds(0, 16)]

            @plsc.parallel_loop(0, _IW, unroll=4)
            def _(ii):
                for h in range(2):
                    w = m_v[slot, rr, pl.ds(ii * 32 + h * 16, 16)]
                    for r in range(4):
                        keep = (w << (31 - 8 * r)) < 0
                        jo = 64 * h + 16 * r
                        v = p_v[slot, rr, ii, pl.ds(jo, 16)]
                        o_v[slot, rr, pl.ds(ii * 128 + jo, 16)] = jnp.where(
                            keep, sbrv - alpha * v, neg_inf)

            return carry

        lax.fori_loop(0, 8, row, 0)

    def _issue_in(rg, cc, slot):
        psrc, msrc, _ = _slices(rg, cc)
        pltpu.async_copy(psrc, p_v.at[slot], sem_in)
        pltpu.async_copy(msrc, m_v.at[slot], sem_in)

    def _wait_in(rg, cc, slot):
        psrc, msrc, _ = _slices(rg, cc)
        pltpu.make_async_copy(psrc, p_v.at[slot], sem_in).wait()
        pltpu.make_async_copy(msrc, m_v.at[slot], sem_in).wait()

    def _issue_out(rg, cc, slot):
        _, _, odst = _slices(rg, cc)
        pltpu.async_copy(o_v.at[slot], odst, sem_out)

    def _wait_out(rg, cc, slot):
        _, _, odst = _slices(rg, cc)
        pltpu.make_async_copy(o_v.at[slot], odst, sem_out).wait()

    for rg_i in range(_RG_PER_W):
        rg = wid * _RG_PER_W + rg_i
        local0 = 8 * rg_i
        _issue_in(rg, 0, 0)

        def step(cc, carry, rg=rg, local0=local0):
            slot = cc % 2

            @pl.when(cc + 1 < _CC)
            def _():
                _issue_in(rg, cc + 1, (cc + 1) % 2)

            _wait_in(rg, cc, slot)

            @pl.when(cc >= 2)
            def _():
                _wait_out(rg, cc - 2, slot)

            _compute(slot, local0)
            _issue_out(rg, cc, slot)
            return carry

        lax.fori_loop(0, _CC, step, 0)
        _wait_out(rg, _CC - 2, 0)
        _wait_out(rg, _CC - 1, 1)


def kernel(pairwise_tti, combined_mask, state_bias, alpha, bias):
    b, t, i, j = pairwise_tti.shape
    m32 = (combined_mask.reshape(b, t, (i * j) // 64, 4, 16)
           .transpose(0, 1, 2, 4, 3)
           .view(jnp.uint8).view(jnp.int32).reshape(b, t, (i * j) // 4))
    sbx = jnp.broadcast_to(
        ((state_bias + bias).reshape(b * t))[:, None, None], (b * t, 1, 16))
    params = jnp.broadcast_to(alpha.reshape(1), (16,))
    mesh = plsc.VectorSubcoreMesh(core_axis_name="c", subcore_axis_name="s",
                                  num_cores=_NC, num_subcores=_NS)
    f = pl.kernel(
        _body,
        out_type=jax.ShapeDtypeStruct((b, t, i * j), jnp.float32),
        mesh=mesh,
        compiler_params=pltpu.CompilerParams(needs_layout_passes=False),
        scratch_types=[
            pltpu.VMEM((2, 8, _IW, 128), jnp.float32),
            pltpu.VMEM((2, 8, _CHUNK // 4), jnp.int32),
            pltpu.VMEM((2, 8, _CHUNK), jnp.float32),
            pltpu.VMEM((32, 1, 16), jnp.float32),
            pltpu.VMEM((16,), jnp.float32),
            pltpu.SemaphoreType.DMA,
            pltpu.SemaphoreType.DMA,
        ],
    )
    return f(pairwise_tti, m32, sbx, params)
